# baseline (speedup 1.0000x reference)
import functools
import numpy as np
import jax
import jax.numpy as jnp
from jax import lax
from jax.experimental import pallas as pl
from jax.experimental.pallas import tpu as pltpu
from jax.experimental.pallas import tpu_sc as plsc

B = 128
C = 100000
CAP = 256
Q = 100
TAU = np.float32(0.95)
INV_T = np.float32(1.0 / 1.3)
LAMDA = np.float32(0.01)
KREG = 5

PEN_TABLE = np.tile(
    np.cumsum(
        (LAMDA * (np.arange(128) >= KREG).astype(np.float32)).astype(np.float32)
    ).astype(np.float32).reshape(1, 128),
    (8, 1),
)

NW = 32
ROWS_PER_W = B // NW
CH = 20000
NCH = C // CH
VPC = CH // 16
LSTRIDE = 4112
HTOT = 16 * LSTRIDE
BIGIDX = np.int32(0x3FFFFFFF)


def _enc_signed(b):
    flip = jnp.bitwise_and(jnp.right_shift(b, 31), jnp.int32(0x7FFFFFFF))
    return jnp.bitwise_xor(b, flip)


SIGNBIT = np.int32(-2147483648)


def _sc_select_body(x_hbm, outv_hbm, outi_hbm,
                    buf0, buf1, hist, cand_v, cand_i, state, sem0, sem1):
    lane = lax.iota(jnp.int32, 16)
    lane_base = lane * LSTRIDE
    ones = jnp.ones((16,), jnp.int32)
    zvec = jnp.zeros((16,), jnp.int32)
    wid = lax.axis_index("s") * 2 + lax.axis_index("c")

    bufs = (buf0, buf1)
    sems = (sem0, sem1)

    def stream(row, per_chunk, carry):
        base = row * C
        handles = {0: pltpu.async_copy(x_hbm.at[pl.ds(base, CH)],
                                       bufs[0], sems[0])}
        for c in range(NCH):
            if c + 1 < NCH:
                handles[c + 1] = pltpu.async_copy(
                    x_hbm.at[pl.ds(base + (c + 1) * CH, CH)],
                    bufs[(c + 1) % 2], sems[(c + 1) % 2])
            handles[c].wait()
            carry = per_chunk(bufs[c % 2], c * CH, carry)
        return carry

    def zero_hist():
        @plsc.parallel_loop(0, HTOT // 16, unroll=8)
        def _(i):
            hist[pl.ds(i * 16, 16)] = zvec

    def hist_pass(row, bucket_fn):
        def pc(buf, off, cr):
            @plsc.parallel_loop(0, VPC, unroll=16)
            def _(i):
                b = plsc.bitcast(buf[pl.ds(i * 16, 16)], jnp.int32)
                bkt, msk = bucket_fn(jnp.bitwise_xor(_enc_signed(b), SIGNBIT))
                addr = lane_base + jnp.bitwise_xor(bkt, lane)
                if msk is None:
                    plsc.addupdate_scatter(hist, [addr], ones)
                else:
                    plsc.addupdate_scatter(hist, [addr], ones, mask=msk)
            return cr
        stream(row, pc, 0)

    def scan_hist(quota):
        def body(g, st):
            carried, found = st
            gb = (255 - g) * 16
            tot = hist[pl.ds(gb, 16)]
            for l in range(1, 16):
                tot = tot + hist[pl.ds(l * LSTRIDE + gb, 16)]
            gsum = jnp.sum(tot)
            hit = (found == 0) & (carried + gsum >= quota)

            @pl.when(hit)
            def _():
                ptot = hist[pl.ds(gb, 16)]
                for l in range(1, 16):
                    vl = hist[pl.ds(l * LSTRIDE + gb, 16)]
                    perm = jnp.bitwise_xor(lane, jnp.int32(l))
                    ptot = ptot + vl.at[perm].get(mode="promise_in_bounds")
                suffix = lax.rev(plsc.cumsum(lax.rev(ptot, (0,))), (0,))
                cumtop = suffix + carried
                mask = cumtop >= quota
                j = jnp.sum(mask.astype(jnp.int32)) - 1
                eq = lane == j
                ct_b = jnp.max(jnp.where(eq, cumtop, 0))
                tot_j = jnp.max(jnp.where(eq, ptot, 0))
                state[3] = gb + j
                state[4] = ct_b - tot_j
                state[5] = tot_j

            found2 = jnp.where(hit, 1, found)
            carried2 = jnp.where(found2 == 1, carried, carried + gsum)
            return (g + 1, carried2, found2)
        lax.while_loop(lambda st: (st[0] < 256) & (st[2] == 0),
                       lambda st: body(st[0], (st[1], st[2])),
                       (0, 0, 0))

    def select_row(rr, _):
        row = wid * ROWS_PER_W + rr

        zero_hist()
        hist_pass(row, lambda m: (
            jnp.bitwise_and(jnp.right_shift(m, 20), jnp.int32(0xFFF)), None))
        scan_hist(Q)
        state[0] = jnp.left_shift(state[3], 20)
        state[1] = state[4]
        state[2] = state[5]

        @pl.when(state[1] + state[2] > CAP)
        def _():
            b1 = state[3]
            zero_hist()

            def bf(m):
                mi = jnp.bitwise_and(jnp.right_shift(m, 20), jnp.int32(0xFFF))
                b2 = jnp.bitwise_and(jnp.right_shift(m, 8), jnp.int32(0xFFF))
                return b2, mi == b1
            hist_pass(row, bf)
            scan_hist(Q - state[1])
            state[0] = jnp.bitwise_or(state[0], jnp.left_shift(state[3], 8))
            state[1] = state[1] + state[4]
            state[2] = state[5]

        @pl.when(state[1] + state[2] > CAP)
        def _():
            pref = jnp.right_shift(state[0], 8)
            zero_hist()

            def bf(m):
                mi24 = jnp.right_shift(m, 8)
                b3 = jnp.bitwise_and(m, jnp.int32(0xFF))
                return b3, mi24 == pref
            hist_pass(row, bf)
            scan_hist(Q - state[1])
            state[0] = jnp.bitwise_or(state[0], state[3])

        neg_inf_bits = jnp.full((16,), -8388608, jnp.int32)
        big = jnp.full((16,), BIGIDX, jnp.int32)
        for k in range(CAP // 16):
            cand_v[pl.ds(k * 16, 16)] = neg_inf_bits
            cand_i[pl.ds(k * 16, 16)] = big
        enc_lo_s = jnp.broadcast_to(
            jnp.bitwise_xor(state[0], SIGNBIT), (16,)).astype(jnp.int32)

        def cc(buf, off, cnt0):
            def cv(i, cnt):
                v = plsc.bitcast(buf[pl.ds(i * 16, 16)], jnp.int32)
                pmask = _enc_signed(v) >= enc_lo_s
                npass = plsc.all_reduce_population_count(pmask)
                pos = plsc.cumsum(pmask.astype(jnp.int32))
                tix = cnt + pos - 1
                smask = pmask & (tix < CAP)
                plsc.store_scatter(cand_v, [tix], v, mask=smask)
                gi = off + i * 16 + lane
                plsc.store_scatter(cand_i, [tix], gi, mask=smask)
                return cnt + npass
            return plsc.parallel_loop(0, VPC, unroll=16, carry=cnt0)(cv)
        stream(row, cc, jnp.zeros((16,), jnp.int32))

        pltpu.sync_copy(cand_v, outv_hbm.at[pl.ds(row * CAP, CAP)])
        pltpu.sync_copy(cand_i, outi_hbm.at[pl.ds(row * CAP, CAP)])
        return 0

    lax.fori_loop(0, ROWS_PER_W, select_row, 0)


_sc_select = functools.partial(
    pl.kernel,
    out_type=(
        jax.ShapeDtypeStruct((B * CAP,), jnp.int32),
        jax.ShapeDtypeStruct((B * CAP,), jnp.int32),
    ),
    scratch_types=[
        pltpu.VMEM((CH,), jnp.float32),
        pltpu.VMEM((CH,), jnp.float32),
        pltpu.VMEM((HTOT,), jnp.int32),
        pltpu.VMEM((CAP,), jnp.int32),
        pltpu.VMEM((CAP,), jnp.int32),
        pltpu.SMEM((8,), jnp.int32),
        pltpu.SemaphoreType.DMA,
        pltpu.SemaphoreType.DMA,
    ],
    mesh=plsc.VectorSubcoreMesh(core_axis_name="c", subcore_axis_name="s"),
    compiler_params=pltpu.CompilerParams(needs_layout_passes=False),
)(_sc_select_body)


WBLK = 4096
NBLK = -(-C // WBLK)


def _tc_mono_i32(x):
    b = lax.bitcast_convert_type(x, jnp.int32)
    flip = jnp.bitwise_and(jnp.right_shift(b, 31), jnp.int32(0x7FFFFFFF))
    return jnp.bitwise_xor(b, flip)


def _stats_body(x_ref, zm_ref, rd_ref, d_ref):
    i = pl.program_id(0)

    @pl.when(i == 0)
    def _():
        zm_ref[...] = jnp.full((B, 1), -jnp.inf, jnp.float32)
        d_ref[...] = jnp.zeros((B, 1), jnp.float32)

    z = x_ref[...] * INV_T
    col = lax.broadcasted_iota(jnp.int32, (B, WBLK), 1) + i * WBLK
    valid = col < C
    bm = jnp.max(jnp.where(valid, z, -jnp.inf), axis=1, keepdims=True)
    m_old = zm_ref[...]
    m_new = jnp.maximum(m_old, bm)
    alpha = jnp.exp(m_old - m_new)
    e = jnp.where(valid, jnp.exp(z - m_new), np.float32(0.0))
    d_new = d_ref[...] * alpha + jnp.sum(e, axis=1, keepdims=True)
    zm_ref[...] = m_new
    d_ref[...] = d_new

    @pl.when(i == NBLK - 1)
    def _():
        rd_ref[...] = np.float32(1.0) / d_new


def _stats(x):
    return pl.pallas_call(
        _stats_body,
        grid=(NBLK,),
        in_specs=[pl.BlockSpec((B, WBLK), lambda i: (0, i))],
        out_specs=[pl.BlockSpec((B, 1), lambda i: (0, 0)),
                   pl.BlockSpec((B, 1), lambda i: (0, 0))],
        out_shape=[jax.ShapeDtypeStruct((B, 1), jnp.float32),
                   jax.ShapeDtypeStruct((B, 1), jnp.float32)],
        scratch_shapes=[pltpu.VMEM((B, 1), jnp.float32)],
    )(x)


RB = 8


def _combine_body(cv_ref, ci_ref, zm_ref, rd_ref, pen_ref,
                  sz_ref, te_ref, ti_ref):
    cv = cv_ref[...]
    ci = ci_ref[...]
    cenc = _tc_mono_i32(cv)
    s = jnp.exp(cv * INV_T - zm_ref[...]) * rd_ref[...]

    e_i = cenc[:, :, None]
    i_i = ci[:, :, None]
    r = jnp.zeros((RB, CAP), jnp.int32)
    S = jnp.zeros((RB, CAP), jnp.float32)
    JC = 128
    for c in range(CAP // JC):
        sl = slice(c * JC, (c + 1) * JC)
        e_j = cenc[:, None, sl]
        i_j = ci[:, None, sl]
        ab = (e_j > e_i) | ((e_j == e_i) & (i_j < i_i))
        r = r + jnp.sum(ab.astype(jnp.int32), axis=2)
        S = S + jnp.sum(jnp.where(ab, s[:, None, sl], np.float32(0.0)), axis=2)

    r_cl = jnp.clip(r, 0, 127)
    k = lax.broadcasted_iota(jnp.int32, (RB, CAP, 128), 2)
    pen = jnp.sum(jnp.where(r_cl[:, :, None] == k, pen_ref[...][:, None, :],
                            np.float32(0.0)), axis=2)
    cond = (S + s + pen) <= TAU
    sizes = 1 + jnp.sum(cond.astype(jnp.int32), axis=1, keepdims=True)
    is_b = r == (sizes - 1)
    sz_ref[...] = sizes
    te_ref[...] = jnp.sum(jnp.where(is_b, cenc, jnp.int32(0)), axis=1,
                          keepdims=True)
    ti_ref[...] = jnp.sum(jnp.where(is_b, ci, jnp.int32(0)), axis=1,
                          keepdims=True)


def _combine(cv, ci, zm, rd, pen):
    return pl.pallas_call(
        _combine_body,
        grid=(B // RB,),
        in_specs=[pl.BlockSpec((RB, CAP), lambda i: (i, 0)),
                  pl.BlockSpec((RB, CAP), lambda i: (i, 0)),
                  pl.BlockSpec((RB, 1), lambda i: (i, 0)),
                  pl.BlockSpec((RB, 1), lambda i: (i, 0)),
                  pl.BlockSpec((RB, 128), lambda i: (0, 0))],
        out_specs=[pl.BlockSpec((RB, 1), lambda i: (i, 0)),
                   pl.BlockSpec((RB, 1), lambda i: (i, 0)),
                   pl.BlockSpec((RB, 1), lambda i: (i, 0))],
        out_shape=[jax.ShapeDtypeStruct((B, 1), jnp.int32),
                   jax.ShapeDtypeStruct((B, 1), jnp.int32),
                   jax.ShapeDtypeStruct((B, 1), jnp.int32)],
    )(cv, ci, zm, rd, pen)


def _mask_body(x_ref, zm_ref, rd_ref, te_ref, ti_ref, o_ref):
    i = pl.program_id(0)
    x = x_ref[...]
    s = jnp.exp(x * INV_T - zm_ref[...]) * rd_ref[...]
    enc = _tc_mono_i32(x)
    col = lax.broadcasted_iota(jnp.int32, (B, WBLK), 1) + i * WBLK
    te = te_ref[...]
    keep = (enc > te) | ((enc == te) & (col <= ti_ref[...]))
    o_ref[...] = jnp.where(keep, s, np.float32(0.0))


def _mask(x, zm, rd, te, ti):
    return pl.pallas_call(
        _mask_body,
        grid=(NBLK,),
        in_specs=[pl.BlockSpec((B, WBLK), lambda i: (0, i)),
                  pl.BlockSpec((B, 1), lambda i: (0, 0)),
                  pl.BlockSpec((B, 1), lambda i: (0, 0)),
                  pl.BlockSpec((B, 1), lambda i: (0, 0)),
                  pl.BlockSpec((B, 1), lambda i: (0, 0))],
        out_specs=pl.BlockSpec((B, WBLK), lambda i: (0, i)),
        out_shape=jax.ShapeDtypeStruct((B, C), jnp.float32),
    )(x, zm, rd, te, ti)


@jax.jit
def kernel(logits):
    cvb_flat, ci_flat = _sc_select(logits.reshape(-1))
    cv = lax.bitcast_convert_type(cvb_flat.reshape(B, CAP), jnp.float32)
    zm, rd = _stats(logits)
    pen = jnp.asarray(PEN_TABLE)
    sz, te, ti = _combine(cv, ci_flat.reshape(B, CAP), zm, rd, pen)
    masked = _mask(logits, zm, rd, te, ti)
    return masked, sz.reshape(B)


# speedup vs baseline: 1.1816x; 1.1816x over previous
import functools
import numpy as np
import jax
import jax.numpy as jnp
from jax import lax
from jax.experimental import pallas as pl
from jax.experimental.pallas import tpu as pltpu
from jax.experimental.pallas import tpu_sc as plsc

B = 128
C = 100000
CAP = 256
Q = 100
TAU = np.float32(0.95)
INV_T = np.float32(1.0 / 1.3)
LAMDA = np.float32(0.01)
KREG = 5

PEN_TABLE = np.tile(
    np.cumsum(
        (LAMDA * (np.arange(128) >= KREG).astype(np.float32)).astype(np.float32)
    ).astype(np.float32).reshape(1, 128),
    (8, 1),
)

NW = 32
ROWS_PER_W = B // NW
CH = 20000
NCH = C // CH
VPC = CH // 16
LSTRIDE = 4112
HTOT = 16 * LSTRIDE
BIGIDX = np.int32(0x3FFFFFFF)


def _enc_signed(b):
    flip = jnp.bitwise_and(jnp.right_shift(b, 31), jnp.int32(0x7FFFFFFF))
    return jnp.bitwise_xor(b, flip)


SIGNBIT = np.int32(-2147483648)


def _sc_select_body(x_hbm, outv_hbm, outi_hbm,
                    buf0, buf1, hist, cand_v, cand_i, state, sem0, sem1):
    lane = lax.iota(jnp.int32, 16)
    lane_base = lane * LSTRIDE
    ones = jnp.ones((16,), jnp.int32)
    zvec = jnp.zeros((16,), jnp.int32)
    wid = lax.axis_index("s") * 2 + lax.axis_index("c")

    bufs = (buf0, buf1)
    sems = (sem0, sem1)

    def stream(row, per_chunk, carry):
        base = row * C
        handles = {0: pltpu.async_copy(x_hbm.at[pl.ds(base, CH)],
                                       bufs[0], sems[0])}
        for c in range(NCH):
            if c + 1 < NCH:
                handles[c + 1] = pltpu.async_copy(
                    x_hbm.at[pl.ds(base + (c + 1) * CH, CH)],
                    bufs[(c + 1) % 2], sems[(c + 1) % 2])
            handles[c].wait()
            carry = per_chunk(bufs[c % 2], c * CH, carry)
        return carry

    def zero_hist():
        @plsc.parallel_loop(0, HTOT // 16, unroll=8)
        def _(i):
            hist[pl.ds(i * 16, 16)] = zvec

    def hist_pass(row, bucket_fn):
        def pc(buf, off, cr):
            @plsc.parallel_loop(0, VPC, unroll=8)
            def _(i):
                b = plsc.bitcast(buf[pl.ds(i * 16, 16)], jnp.int32)
                bkt, msk = bucket_fn(jnp.bitwise_xor(_enc_signed(b), SIGNBIT))
                addr = lane_base + jnp.bitwise_xor(bkt, lane)
                if msk is None:
                    plsc.addupdate_scatter(hist, [addr], ones)
                else:
                    plsc.addupdate_scatter(hist, [addr], ones, mask=msk)
            return cr
        stream(row, pc, 0)

    def scan_hist(quota):
        def body(g, st):
            carried, found = st
            gb = (255 - g) * 16
            tot = hist[pl.ds(gb, 16)]
            for l in range(1, 16):
                tot = tot + hist[pl.ds(l * LSTRIDE + gb, 16)]
            gsum = jnp.sum(tot)
            hit = (found == 0) & (carried + gsum >= quota)

            @pl.when(hit)
            def _():
                ptot = hist[pl.ds(gb, 16)]
                for l in range(1, 16):
                    vl = hist[pl.ds(l * LSTRIDE + gb, 16)]
                    perm = jnp.bitwise_xor(lane, jnp.int32(l))
                    ptot = ptot + vl.at[perm].get(mode="promise_in_bounds")
                suffix = lax.rev(plsc.cumsum(lax.rev(ptot, (0,))), (0,))
                cumtop = suffix + carried
                mask = cumtop >= quota
                j = jnp.sum(mask.astype(jnp.int32)) - 1
                eq = lane == j
                ct_b = jnp.max(jnp.where(eq, cumtop, 0))
                tot_j = jnp.max(jnp.where(eq, ptot, 0))
                state[3] = gb + j
                state[4] = ct_b - tot_j
                state[5] = tot_j

            found2 = jnp.where(hit, 1, found)
            carried2 = jnp.where(found2 == 1, carried, carried + gsum)
            return (g + 1, carried2, found2)
        lax.while_loop(lambda st: (st[0] < 256) & (st[2] == 0),
                       lambda st: body(st[0], (st[1], st[2])),
                       (0, 0, 0))

    def select_row(rr, _):
        row = wid * ROWS_PER_W + rr

        zero_hist()
        hist_pass(row, lambda m: (
            jnp.bitwise_and(jnp.right_shift(m, 20), jnp.int32(0xFFF)), None))
        scan_hist(Q)
        state[0] = jnp.left_shift(state[3], 20)
        state[1] = state[4]
        state[2] = state[5]

        @pl.when(state[1] + state[2] > CAP)
        def _():
            b1 = state[3]
            zero_hist()

            def bf(m):
                mi = jnp.bitwise_and(jnp.right_shift(m, 20), jnp.int32(0xFFF))
                b2 = jnp.bitwise_and(jnp.right_shift(m, 8), jnp.int32(0xFFF))
                return b2, mi == b1
            hist_pass(row, bf)
            scan_hist(Q - state[1])
            state[0] = jnp.bitwise_or(state[0], jnp.left_shift(state[3], 8))
            state[1] = state[1] + state[4]
            state[2] = state[5]

        @pl.when(state[1] + state[2] > CAP)
        def _():
            pref = jnp.right_shift(state[0], 8)
            zero_hist()

            def bf(m):
                mi24 = jnp.right_shift(m, 8)
                b3 = jnp.bitwise_and(m, jnp.int32(0xFF))
                return b3, mi24 == pref
            hist_pass(row, bf)
            scan_hist(Q - state[1])
            state[0] = jnp.bitwise_or(state[0], state[3])

        neg_inf_bits = jnp.full((16,), -8388608, jnp.int32)
        big = jnp.full((16,), BIGIDX, jnp.int32)
        for k in range(CAP // 16):
            cand_v[pl.ds(k * 16, 16)] = neg_inf_bits
            cand_i[pl.ds(k * 16, 16)] = big
        enc_lo_s = jnp.broadcast_to(
            jnp.bitwise_xor(state[0], SIGNBIT), (16,)).astype(jnp.int32)

        def cc(buf, off, cnt0):
            def cv(i, cnt):
                v = plsc.bitcast(buf[pl.ds(i * 16, 16)], jnp.int32)
                pmask = _enc_signed(v) >= enc_lo_s
                npass = plsc.all_reduce_population_count(pmask)
                pos = plsc.cumsum(pmask.astype(jnp.int32))
                tix = cnt + pos - 1
                smask = pmask & (tix < CAP)
                plsc.store_scatter(cand_v, [tix], v, mask=smask)
                gi = off + i * 16 + lane
                plsc.store_scatter(cand_i, [tix], gi, mask=smask)
                return cnt + npass
            return plsc.parallel_loop(0, VPC, unroll=8, carry=cnt0)(cv)
        stream(row, cc, jnp.zeros((16,), jnp.int32))

        pltpu.sync_copy(cand_v, outv_hbm.at[pl.ds(row * CAP, CAP)])
        pltpu.sync_copy(cand_i, outi_hbm.at[pl.ds(row * CAP, CAP)])
        return 0

    lax.fori_loop(0, ROWS_PER_W, select_row, 0)


_sc_select = functools.partial(
    pl.kernel,
    out_type=(
        jax.ShapeDtypeStruct((B * CAP,), jnp.int32),
        jax.ShapeDtypeStruct((B * CAP,), jnp.int32),
    ),
    scratch_types=[
        pltpu.VMEM((CH,), jnp.float32),
        pltpu.VMEM((CH,), jnp.float32),
        pltpu.VMEM((HTOT,), jnp.int32),
        pltpu.VMEM((CAP,), jnp.int32),
        pltpu.VMEM((CAP,), jnp.int32),
        pltpu.SMEM((8,), jnp.int32),
        pltpu.SemaphoreType.DMA,
        pltpu.SemaphoreType.DMA,
    ],
    mesh=plsc.VectorSubcoreMesh(core_axis_name="c", subcore_axis_name="s"),
    compiler_params=pltpu.CompilerParams(needs_layout_passes=False),
)(_sc_select_body)


WBLK = 4096
NBLK = -(-C // WBLK)


def _tc_mono_i32(x):
    b = lax.bitcast_convert_type(x, jnp.int32)
    flip = jnp.bitwise_and(jnp.right_shift(b, 31), jnp.int32(0x7FFFFFFF))
    return jnp.bitwise_xor(b, flip)


def _stats_body(x_ref, zm_ref, rd_ref, d_ref):
    i = pl.program_id(0)

    @pl.when(i == 0)
    def _():
        zm_ref[...] = jnp.full((B, 1), -jnp.inf, jnp.float32)
        d_ref[...] = jnp.zeros((B, 1), jnp.float32)

    z = x_ref[...] * INV_T
    col = lax.broadcasted_iota(jnp.int32, (B, WBLK), 1) + i * WBLK
    valid = col < C
    bm = jnp.max(jnp.where(valid, z, -jnp.inf), axis=1, keepdims=True)
    m_old = zm_ref[...]
    m_new = jnp.maximum(m_old, bm)
    alpha = jnp.exp(m_old - m_new)
    e = jnp.where(valid, jnp.exp(z - m_new), np.float32(0.0))
    d_new = d_ref[...] * alpha + jnp.sum(e, axis=1, keepdims=True)
    zm_ref[...] = m_new
    d_ref[...] = d_new

    @pl.when(i == NBLK - 1)
    def _():
        rd_ref[...] = np.float32(1.0) / d_new


def _stats(x):
    return pl.pallas_call(
        _stats_body,
        grid=(NBLK,),
        in_specs=[pl.BlockSpec((B, WBLK), lambda i: (0, i))],
        out_specs=[pl.BlockSpec((B, 1), lambda i: (0, 0)),
                   pl.BlockSpec((B, 1), lambda i: (0, 0))],
        out_shape=[jax.ShapeDtypeStruct((B, 1), jnp.float32),
                   jax.ShapeDtypeStruct((B, 1), jnp.float32)],
        scratch_shapes=[pltpu.VMEM((B, 1), jnp.float32)],
    )(x)


RB = 8


def _combine_body(cv_ref, ci_ref, zm_ref, rd_ref, pen_ref,
                  sz_ref, te_ref, ti_ref):
    cv = cv_ref[...]
    ci = ci_ref[...]
    cenc = _tc_mono_i32(cv)
    s = jnp.exp(cv * INV_T - zm_ref[...]) * rd_ref[...]

    e_i = cenc[:, :, None]
    i_i = ci[:, :, None]
    r = jnp.zeros((RB, CAP), jnp.int32)
    S = jnp.zeros((RB, CAP), jnp.float32)
    JC = 128
    for c in range(CAP // JC):
        sl = slice(c * JC, (c + 1) * JC)
        e_j = cenc[:, None, sl]
        i_j = ci[:, None, sl]
        ab = (e_j > e_i) | ((e_j == e_i) & (i_j < i_i))
        r = r + jnp.sum(ab.astype(jnp.int32), axis=2)
        S = S + jnp.sum(jnp.where(ab, s[:, None, sl], np.float32(0.0)), axis=2)

    r_cl = jnp.clip(r, 0, 127)
    k = lax.broadcasted_iota(jnp.int32, (RB, CAP, 128), 2)
    pen = jnp.sum(jnp.where(r_cl[:, :, None] == k, pen_ref[...][:, None, :],
                            np.float32(0.0)), axis=2)
    cond = (S + s + pen) <= TAU
    sizes = 1 + jnp.sum(cond.astype(jnp.int32), axis=1, keepdims=True)
    is_b = r == (sizes - 1)
    sz_ref[...] = sizes
    te_ref[...] = jnp.sum(jnp.where(is_b, cenc, jnp.int32(0)), axis=1,
                          keepdims=True)
    ti_ref[...] = jnp.sum(jnp.where(is_b, ci, jnp.int32(0)), axis=1,
                          keepdims=True)


def _combine(cv, ci, zm, rd, pen):
    return pl.pallas_call(
        _combine_body,
        grid=(B // RB,),
        in_specs=[pl.BlockSpec((RB, CAP), lambda i: (i, 0)),
                  pl.BlockSpec((RB, CAP), lambda i: (i, 0)),
                  pl.BlockSpec((RB, 1), lambda i: (i, 0)),
                  pl.BlockSpec((RB, 1), lambda i: (i, 0)),
                  pl.BlockSpec((RB, 128), lambda i: (0, 0))],
        out_specs=[pl.BlockSpec((RB, 1), lambda i: (i, 0)),
                   pl.BlockSpec((RB, 1), lambda i: (i, 0)),
                   pl.BlockSpec((RB, 1), lambda i: (i, 0))],
        out_shape=[jax.ShapeDtypeStruct((B, 1), jnp.int32),
                   jax.ShapeDtypeStruct((B, 1), jnp.int32),
                   jax.ShapeDtypeStruct((B, 1), jnp.int32)],
    )(cv, ci, zm, rd, pen)


def _mask_body(x_ref, zm_ref, rd_ref, te_ref, ti_ref, o_ref):
    i = pl.program_id(0)
    x = x_ref[...]
    s = jnp.exp(x * INV_T - zm_ref[...]) * rd_ref[...]
    enc = _tc_mono_i32(x)
    col = lax.broadcasted_iota(jnp.int32, (B, WBLK), 1) + i * WBLK
    te = te_ref[...]
    keep = (enc > te) | ((enc == te) & (col <= ti_ref[...]))
    o_ref[...] = jnp.where(keep, s, np.float32(0.0))


def _mask(x, zm, rd, te, ti):
    return pl.pallas_call(
        _mask_body,
        grid=(NBLK,),
        in_specs=[pl.BlockSpec((B, WBLK), lambda i: (0, i)),
                  pl.BlockSpec((B, 1), lambda i: (0, 0)),
                  pl.BlockSpec((B, 1), lambda i: (0, 0)),
                  pl.BlockSpec((B, 1), lambda i: (0, 0)),
                  pl.BlockSpec((B, 1), lambda i: (0, 0))],
        out_specs=pl.BlockSpec((B, WBLK), lambda i: (0, i)),
        out_shape=jax.ShapeDtypeStruct((B, C), jnp.float32),
    )(x, zm, rd, te, ti)


@jax.jit
def kernel(logits):
    cvb_flat, ci_flat = _sc_select(logits.reshape(-1))
    cv = lax.bitcast_convert_type(cvb_flat.reshape(B, CAP), jnp.float32)
    zm, rd = _stats(logits)
    pen = jnp.asarray(PEN_TABLE)
    sz, te, ti = _combine(cv, ci_flat.reshape(B, CAP), zm, rd, pen)
    masked = _mask(logits, zm, rd, te, ti)
    return masked, sz.reshape(B)
